# baseline (speedup 1.0000x reference)
import jax
import jax.numpy as jnp
from jax.experimental import pallas as pl
from jax.experimental.pallas import tpu as pltpu

_BLK = 1024


def _attn_kernel(x_ref, wq_ref, wk_ref, wv_ref, sha_ref, ent_ref):
    x = x_ref[0]
    T = x.shape[0]
    q = jnp.dot(x, wq_ref[0], preferred_element_type=jnp.float32)
    k = jnp.dot(x, wk_ref[0], preferred_element_type=jnp.float32)
    v = jnp.dot(x, wv_ref[0], preferred_element_type=jnp.float32)
    q = q * (1.0 / (q.shape[-1] ** 0.5))

    ent_total = jnp.float32(0.0)
    for i in range(T // _BLK):
        qb = q[i * _BLK:(i + 1) * _BLK, :]
        s = jax.lax.dot_general(
            qb, k, (((1,), (1,)), ((), ())),
            preferred_element_type=jnp.float32)
        mx = jnp.max(s, axis=-1, keepdims=True)
        ex = jnp.exp(s - mx)
        z = jnp.sum(ex, axis=-1, keepdims=True)
        exs = jnp.sum(ex * s, axis=-1, keepdims=True)
        ent_total += jnp.sum(jnp.log(z) + mx - exs / z)
        ob = jnp.dot(ex, v, preferred_element_type=jnp.float32) / z
        sha_ref[0, 0, i * _BLK:(i + 1) * _BLK, :] = ob
    ent_ref[0] = jnp.full((1, 128), ent_total / T, dtype=jnp.float32)


def _gate_kernel(ent_ref, g_ref, logits_ref, mask_ref, nmask_ref, fb_ref):
    ent = ent_ref[...]
    Bv, Ev = ent.shape
    aff = -ent
    mean = jnp.mean(aff, axis=-1, keepdims=True)
    var = jnp.sum((aff - mean) ** 2, axis=-1, keepdims=True) / (Ev - 1)
    std = jnp.sqrt(var)
    affn = (aff - mean) / (std + 1e-9)
    logits = affn - jax.nn.sigmoid(g_ref[...])
    hard = (logits > 0).astype(jnp.float32)
    num_active = jnp.sum(hard, axis=1, keepdims=True)
    inactive = num_active == 0.0
    cols = jax.lax.broadcasted_iota(jnp.int32, (Bv, Ev), 1)
    rowmax = jnp.max(affn, axis=1, keepdims=True)
    first = jnp.min(jnp.where(affn >= rowmax, cols, Ev), axis=1, keepdims=True)
    fb_onehot = (cols == first).astype(jnp.float32)
    mask = jnp.where(inactive, jnp.maximum(hard, fb_onehot), hard)
    na2 = jnp.sum(mask, axis=1, keepdims=True)
    nmask = mask / jnp.clip(na2, 1.0, None)
    logits_ref[...] = logits
    mask_ref[...] = mask
    nmask_ref[...] = nmask
    fb_ref[...] = jnp.sum(inactive.astype(jnp.int32)).reshape(1, 1)


def _combine_kernel(sha_ref, nmask_ref, o_ref, out_ref, sha_t_ref):
    w = nmask_ref[0, 0]
    sha = sha_ref[0]
    sha_t_ref[0] = jnp.transpose(sha, (1, 0, 2))
    combined = jnp.sum(sha * w[:, None, None], axis=0)
    oproj = jnp.sum(o_ref[...] * w[:, None, None], axis=0)
    out_ref[0] = jnp.dot(combined, oproj, preferred_element_type=jnp.float32)


def kernel(hidden_states, Wq, Wk, Wv, gates, o_weights):
    B, T, D = hidden_states.shape
    E, _, H = Wq.shape

    sha_beth, ent_raw = pl.pallas_call(
        _attn_kernel,
        grid=(B * E,),
        in_specs=[
            pl.BlockSpec((1, T, D), lambda i: (i // E, 0, 0)),
            pl.BlockSpec((1, D, H), lambda i: (i % E, 0, 0)),
            pl.BlockSpec((1, D, H), lambda i: (i % E, 0, 0)),
            pl.BlockSpec((1, D, H), lambda i: (i % E, 0, 0)),
        ],
        out_specs=[
            pl.BlockSpec((1, 1, T, H), lambda i: (i // E, i % E, 0, 0)),
            pl.BlockSpec((1, 1, 128), lambda i: (i, 0, 0)),
        ],
        out_shape=[
            jax.ShapeDtypeStruct((B, E, T, H), jnp.float32),
            jax.ShapeDtypeStruct((B * E, 1, 128), jnp.float32),
        ],
    )(hidden_states, Wq, Wk, Wv)

    mean_entropy = ent_raw[:, 0, 0].reshape(B, E)
    gates2d = gates.reshape(1, E)

    logits, mask, nmask, fb = pl.pallas_call(
        _gate_kernel,
        in_specs=[
            pl.BlockSpec((B, E), lambda: (0, 0)),
            pl.BlockSpec((1, E), lambda: (0, 0)),
        ],
        out_specs=[
            pl.BlockSpec((B, E), lambda: (0, 0)),
            pl.BlockSpec((B, E), lambda: (0, 0)),
            pl.BlockSpec((B, E), lambda: (0, 0)),
            pl.BlockSpec((1, 1), lambda: (0, 0)),
        ],
        out_shape=[
            jax.ShapeDtypeStruct((B, E), jnp.float32),
            jax.ShapeDtypeStruct((B, E), jnp.float32),
            jax.ShapeDtypeStruct((B, E), jnp.float32),
            jax.ShapeDtypeStruct((1, 1), jnp.int32),
        ],
    )(mean_entropy, gates2d)

    final, all_sha_outputs = pl.pallas_call(
        _combine_kernel,
        grid=(B,),
        in_specs=[
            pl.BlockSpec((1, E, T, H), lambda b: (b, 0, 0, 0)),
            pl.BlockSpec((1, 1, E), lambda b: (b, 0, 0)),
            pl.BlockSpec((E, H, D), lambda b: (0, 0, 0)),
        ],
        out_specs=[
            pl.BlockSpec((1, T, D), lambda b: (b, 0, 0)),
            pl.BlockSpec((1, T, E, H), lambda b: (b, 0, 0, 0)),
        ],
        out_shape=[
            jax.ShapeDtypeStruct((B, T, D), jnp.float32),
            jax.ShapeDtypeStruct((B, T, E, H), jnp.float32),
        ],
    )(sha_beth, nmask.reshape(B, 1, E), o_weights)
    fallback_count = fb.reshape(()).astype(jnp.int32)
    return final, all_sha_outputs, logits, mask, fallback_count


# speedup vs baseline: 1.1097x; 1.1097x over previous
import jax
import jax.numpy as jnp
from jax.experimental import pallas as pl
from jax.experimental.pallas import tpu as pltpu

_BLK = 512


def _attn_kernel(x_ref, wq_ref, wk_ref, wv_ref, sha_ref, ent_ref):
    x = x_ref[0]
    T = x.shape[0]
    q = jnp.dot(x, wq_ref[0], preferred_element_type=jnp.float32)
    k = jnp.dot(x, wk_ref[0], preferred_element_type=jnp.float32)
    v = jnp.dot(x, wv_ref[0], preferred_element_type=jnp.float32)
    q = q * (1.0 / (q.shape[-1] ** 0.5))

    ent_total = jnp.float32(0.0)
    for i in range(T // _BLK):
        qb = q[i * _BLK:(i + 1) * _BLK, :]
        s = jax.lax.dot_general(
            qb, k, (((1,), (1,)), ((), ())),
            preferred_element_type=jnp.float32)
        mx = jnp.max(s, axis=-1, keepdims=True)
        ex = jnp.exp(s - mx)
        z = jnp.sum(ex, axis=-1, keepdims=True)
        exs = jnp.sum(ex * s, axis=-1, keepdims=True)
        ent_total += jnp.sum(jnp.log(z) + mx - exs / z)
        ob = jnp.dot(ex, v, preferred_element_type=jnp.float32) / z
        sha_ref[0, 0, i * _BLK:(i + 1) * _BLK, :] = ob
    ent_ref[0] = jnp.full((1, 128), ent_total / T, dtype=jnp.float32)


def _gate_kernel(ent_ref, g_ref, logits_ref, mask_ref, nmask_ref, fb_ref):
    ent = ent_ref[...]
    Bv, Ev = ent.shape
    aff = -ent
    mean = jnp.mean(aff, axis=-1, keepdims=True)
    var = jnp.sum((aff - mean) ** 2, axis=-1, keepdims=True) / (Ev - 1)
    std = jnp.sqrt(var)
    affn = (aff - mean) / (std + 1e-9)
    logits = affn - jax.nn.sigmoid(g_ref[...])
    hard = (logits > 0).astype(jnp.float32)
    num_active = jnp.sum(hard, axis=1, keepdims=True)
    inactive = num_active == 0.0
    cols = jax.lax.broadcasted_iota(jnp.int32, (Bv, Ev), 1)
    rowmax = jnp.max(affn, axis=1, keepdims=True)
    first = jnp.min(jnp.where(affn >= rowmax, cols, Ev), axis=1, keepdims=True)
    fb_onehot = (cols == first).astype(jnp.float32)
    mask = jnp.where(inactive, jnp.maximum(hard, fb_onehot), hard)
    na2 = jnp.sum(mask, axis=1, keepdims=True)
    nmask = mask / jnp.clip(na2, 1.0, None)
    logits_ref[...] = logits
    mask_ref[...] = mask
    nmask_ref[...] = nmask
    fb_ref[...] = jnp.sum(inactive.astype(jnp.int32)).reshape(1, 1)


def _combine_kernel(sha_ref, nmask_ref, o_ref, out_ref, sha_t_ref):
    w = nmask_ref[0, 0]
    sha = sha_ref[0]
    sha_t_ref[0] = jnp.transpose(sha, (1, 0, 2))
    combined = jnp.sum(sha * w[:, None, None], axis=0)
    oproj = jnp.sum(o_ref[...] * w[:, None, None], axis=0)
    out_ref[0] = jnp.dot(combined, oproj, preferred_element_type=jnp.float32)


def kernel(hidden_states, Wq, Wk, Wv, gates, o_weights):
    B, T, D = hidden_states.shape
    E, _, H = Wq.shape

    sha_beth, ent_raw = pl.pallas_call(
        _attn_kernel,
        grid=(B * E,),
        in_specs=[
            pl.BlockSpec((1, T, D), lambda i: (i // E, 0, 0)),
            pl.BlockSpec((1, D, H), lambda i: (i % E, 0, 0)),
            pl.BlockSpec((1, D, H), lambda i: (i % E, 0, 0)),
            pl.BlockSpec((1, D, H), lambda i: (i % E, 0, 0)),
        ],
        out_specs=[
            pl.BlockSpec((1, 1, T, H), lambda i: (i // E, i % E, 0, 0)),
            pl.BlockSpec((1, 1, 128), lambda i: (i, 0, 0)),
        ],
        out_shape=[
            jax.ShapeDtypeStruct((B, E, T, H), jnp.float32),
            jax.ShapeDtypeStruct((B * E, 1, 128), jnp.float32),
        ],
    )(hidden_states, Wq, Wk, Wv)

    mean_entropy = ent_raw[:, 0, 0].reshape(B, E)
    gates2d = gates.reshape(1, E)

    logits, mask, nmask, fb = pl.pallas_call(
        _gate_kernel,
        in_specs=[
            pl.BlockSpec((B, E), lambda: (0, 0)),
            pl.BlockSpec((1, E), lambda: (0, 0)),
        ],
        out_specs=[
            pl.BlockSpec((B, E), lambda: (0, 0)),
            pl.BlockSpec((B, E), lambda: (0, 0)),
            pl.BlockSpec((B, E), lambda: (0, 0)),
            pl.BlockSpec((1, 1), lambda: (0, 0)),
        ],
        out_shape=[
            jax.ShapeDtypeStruct((B, E), jnp.float32),
            jax.ShapeDtypeStruct((B, E), jnp.float32),
            jax.ShapeDtypeStruct((B, E), jnp.float32),
            jax.ShapeDtypeStruct((1, 1), jnp.int32),
        ],
    )(mean_entropy, gates2d)

    final, all_sha_outputs = pl.pallas_call(
        _combine_kernel,
        grid=(B,),
        in_specs=[
            pl.BlockSpec((1, E, T, H), lambda b: (b, 0, 0, 0)),
            pl.BlockSpec((1, 1, E), lambda b: (b, 0, 0)),
            pl.BlockSpec((E, H, D), lambda b: (0, 0, 0)),
        ],
        out_specs=[
            pl.BlockSpec((1, T, D), lambda b: (b, 0, 0)),
            pl.BlockSpec((1, T, E, H), lambda b: (b, 0, 0, 0)),
        ],
        out_shape=[
            jax.ShapeDtypeStruct((B, T, D), jnp.float32),
            jax.ShapeDtypeStruct((B, T, E, H), jnp.float32),
        ],
    )(sha_beth, nmask.reshape(B, 1, E), o_weights)
    fallback_count = fb.reshape(()).astype(jnp.int32)
    return final, all_sha_outputs, logits, mask, fallback_count
